# baseline (speedup 1.0000x reference)
import functools

import jax
import jax.numpy as jnp
from jax.experimental import pallas as pl
from jax.sharding import Mesh, PartitionSpec as P

try:
    from jax import shard_map as _shard_map
except ImportError:
    from jax.experimental.shard_map import shard_map as _shard_map

_NUM_EXPERTS = 64
_TOP_K = 8
_BLK = 1024


def _router_kernel(x_ref, w_ref, wout_ref, iout_ref, acc_ref, *, precision):
    i = pl.program_id(0)

    @pl.when(i == 0)
    def _init():
        acc_ref[...] = jnp.zeros_like(acc_ref)

    x = x_ref[...]
    w = w_ref[...]
    logits_t = jax.lax.dot_general(
        w, x, (((1,), (1,)), ((), ())),
        precision=precision,
        preferred_element_type=jnp.float32)

    m = jnp.max(logits_t, axis=0, keepdims=True)
    e = jnp.exp(logits_t - m)
    s = jnp.sum(e, axis=0, keepdims=True)
    probs = e / s

    iota = jax.lax.broadcasted_iota(jnp.int32, probs.shape, 0)
    p = probs
    sel = jnp.zeros_like(probs)
    ws = []
    idxs = []
    for _ in range(_TOP_K):
        mx = jnp.max(p, axis=0, keepdims=True)
        is_max = p == mx
        idx = jnp.min(jnp.where(is_max, iota, _NUM_EXPERTS),
                      axis=0, keepdims=True)
        chosen = iota == idx
        p = jnp.where(chosen, -1.0, p)
        sel = sel + chosen.astype(jnp.float32)
        ws.append(mx)
        idxs.append(idx)

    topw = jnp.concatenate(ws, axis=0)
    wout_ref[...] = topw / jnp.sum(topw, axis=0, keepdims=True)
    iout_ref[...] = jnp.concatenate(idxs, axis=0)

    acc_ref[:, 0:1] += jnp.sum(sel, axis=1, keepdims=True)
    acc_ref[:, 1:2] += jnp.sum(probs, axis=1, keepdims=True)


def _router_shard(hidden_states, gate_weight, *, precision):
    num_tokens, hid = hidden_states.shape
    nsteps = num_tokens // _BLK
    body = functools.partial(_router_kernel, precision=precision)
    return pl.pallas_call(
        body,
        grid=(nsteps,),
        in_specs=[
            pl.BlockSpec((_BLK, hid), lambda i: (i, 0)),
            pl.BlockSpec((_NUM_EXPERTS, hid), lambda i: (0, 0)),
        ],
        out_specs=[
            pl.BlockSpec((_TOP_K, _BLK), lambda i: (0, i)),
            pl.BlockSpec((_TOP_K, _BLK), lambda i: (0, i)),
            pl.BlockSpec((_NUM_EXPERTS, 2), lambda i: (0, 0)),
        ],
        out_shape=[
            jax.ShapeDtypeStruct((_TOP_K, num_tokens), jnp.float32),
            jax.ShapeDtypeStruct((_TOP_K, num_tokens), jnp.int32),
            jax.ShapeDtypeStruct((_NUM_EXPERTS, 2), jnp.float32),
        ],
    )(hidden_states, gate_weight)


@jax.jit
def kernel(hidden_states, gate_weight):
    num_tokens = hidden_states.shape[0]
    devs = jax.devices()
    ndev = len(devs) if num_tokens % (len(devs) * _BLK) == 0 else 1
    shard_fn = functools.partial(_router_shard,
                                 precision=jax.lax.Precision.DEFAULT)
    mesh = Mesh(devs[:ndev], ("d",))
    wout_t, iout_t, accs = _shard_map(
        shard_fn, mesh=mesh,
        in_specs=(P("d", None), P(None, None)),
        out_specs=(P(None, "d"), P(None, "d"), P("d", None)),
        check_vma=False,
    )(hidden_states, gate_weight)
    acc = accs.reshape(ndev, _NUM_EXPERTS, 2).sum(axis=0)
    scale = _NUM_EXPERTS / (float(num_tokens) * float(num_tokens))
    aux = scale * jnp.sum(acc[:, 0] * acc[:, 1])
    return wout_t.T, iout_t.T, aux


# speedup vs baseline: 6.9394x; 6.9394x over previous
import functools

import jax
import jax.numpy as jnp
from jax.experimental import pallas as pl
from jax.experimental.pallas import tpu as pltpu

_NUM_EXPERTS = 64
_TOP_K = 8
_BLK = 1024


def _router_kernel(x_ref, w_ref, wout_ref, iout_ref, aux_ref, acc_ref,
                   *, num_tokens, precision):
    i = pl.program_id(0)
    nsteps = pl.num_programs(0)

    @pl.when(i == 0)
    def _init():
        acc_ref[...] = jnp.zeros_like(acc_ref)

    x = x_ref[...]
    w = w_ref[...]
    logits_t = jax.lax.dot_general(
        w, x, (((1,), (1,)), ((), ())),
        precision=precision,
        preferred_element_type=jnp.float32)

    m = jnp.max(logits_t, axis=0, keepdims=True)
    e = jnp.exp(logits_t - m)
    s = jnp.sum(e, axis=0, keepdims=True)
    probs = e / s

    iota = jax.lax.broadcasted_iota(jnp.int32, probs.shape, 0)
    p = probs
    sel = jnp.zeros_like(probs)
    ws = []
    idxs = []
    for _ in range(_TOP_K):
        mx = jnp.max(p, axis=0, keepdims=True)
        is_max = p == mx
        idx = jnp.min(jnp.where(is_max, iota, _NUM_EXPERTS),
                      axis=0, keepdims=True)
        chosen = iota == idx
        p = jnp.where(chosen, -1.0, p)
        sel = sel + chosen.astype(jnp.float32)
        ws.append(mx)
        idxs.append(idx)

    topw = jnp.concatenate(ws, axis=0)
    wout_ref[...] = topw / jnp.sum(topw, axis=0, keepdims=True)
    iout_ref[...] = jnp.concatenate(idxs, axis=0)

    acc_ref[:, 0:1] += jnp.sum(sel, axis=1, keepdims=True)
    acc_ref[:, 1:2] += jnp.sum(probs, axis=1, keepdims=True)

    @pl.when(i == nsteps - 1)
    def _finish():
        prod = acc_ref[:, 0:1] * acc_ref[:, 1:2]
        scale = _NUM_EXPERTS / (float(num_tokens) * float(num_tokens))
        aux_ref[...] = scale * jnp.sum(prod, axis=0, keepdims=True)


@jax.jit
def kernel(hidden_states, gate_weight):
    num_tokens, hid = hidden_states.shape
    nsteps = num_tokens // _BLK
    body = functools.partial(_router_kernel, num_tokens=num_tokens,
                             precision=jax.lax.Precision.DEFAULT)
    wout_t, iout_t, aux = pl.pallas_call(
        body,
        grid=(nsteps,),
        in_specs=[
            pl.BlockSpec((_BLK, hid), lambda i: (i, 0)),
            pl.BlockSpec((_NUM_EXPERTS, hid), lambda i: (0, 0)),
        ],
        out_specs=[
            pl.BlockSpec((_TOP_K, _BLK), lambda i: (0, i)),
            pl.BlockSpec((_TOP_K, _BLK), lambda i: (0, i)),
            pl.BlockSpec((1, 1), lambda i: (0, 0)),
        ],
        out_shape=[
            jax.ShapeDtypeStruct((_TOP_K, num_tokens), jnp.float32),
            jax.ShapeDtypeStruct((_TOP_K, num_tokens), jnp.int32),
            jax.ShapeDtypeStruct((1, 1), jnp.float32),
        ],
        scratch_shapes=[pltpu.VMEM((_NUM_EXPERTS, 2), jnp.float32)],
    )(hidden_states, gate_weight)
    return wout_t.T, iout_t.T, aux[0, 0]
